# baseline (speedup 1.0000x reference)
import functools
import math

import numpy as np
import jax
import jax.numpy as jnp
from jax import lax
from jax.experimental import pallas as pl
from jax.experimental.pallas import tpu as pltpu
from jax.experimental.pallas import tpu_sc as plsc

_VOCAB = 50000
_D = 768
_H = 12
_DH = 64
_S = 2048
_B = 2
_NHASH = 4
_BKT = 64
_NBKT = 32
_NCLS = 50
_N = _B * _H
_G = _N * _NHASH
_NCHUNK = _S // _BKT
_NW = 32


def _pe_table():
    pos = np.arange(_S)[:, None].astype(np.float32)
    div = np.exp(np.arange(0, _D, 2).astype(np.float32) * (-np.log(10000.0) / _D))
    pe = np.zeros((_S, _D), dtype=np.float32)
    pe[:, 0::2] = np.sin(pos * div)
    pe[:, 1::2] = np.cos(pos * div)
    return pe

_PE = _pe_table()


def _sc_gather_rows(table, idx, chunk=128):
    nrow, d = table.shape
    (ng,) = idx.shape
    per_w = ng // _NW
    chunk = min(chunk, per_w)
    n_ch = per_w // chunk
    assert per_w % chunk == 0 and ng % _NW == 0
    nbuf = min(3, n_ch)
    idx2 = idx.reshape(_NW * n_ch, chunk)
    mesh = plsc.VectorSubcoreMesh(core_axis_name="c", subcore_axis_name="s")

    @functools.partial(
        pl.kernel, mesh=mesh,
        out_type=jax.ShapeDtypeStruct((ng, d), jnp.float32),
        scratch_types=(
            [pltpu.VMEM((n_ch, chunk), jnp.int32)]
            + [pltpu.VMEM((chunk, d), jnp.float32)] * nbuf
            + [pltpu.SemaphoreType.DMA] * (2 * nbuf)),
    )
    def k(table_hbm, idx_hbm, out_hbm, idx_all, *bufs_sems):
        rb = bufs_sems[:nbuf]
        gs = bufs_sems[nbuf:2 * nbuf]
        ss = bufs_sems[2 * nbuf:]
        wid = lax.axis_index("s") * 2 + lax.axis_index("c")
        base = wid * per_w
        pltpu.sync_copy(idx_hbm.at[pl.ds(wid * n_ch, n_ch)], idx_all)
        gathers = [None] * n_ch
        stores = [None] * n_ch
        for c in range(n_ch):
            bi = c % nbuf
            if c - nbuf >= 0:
                stores[c - nbuf].wait()
            gathers[c] = pltpu.async_copy(
                table_hbm.at[idx_all.at[c]], rb[bi], gs[bi])
            if c >= 1:
                gathers[c - 1].wait()
                stores[c - 1] = pltpu.async_copy(
                    rb[(c - 1) % nbuf],
                    out_hbm.at[pl.ds(base + (c - 1) * chunk, chunk)],
                    ss[(c - 1) % nbuf])
        gathers[n_ch - 1].wait()
        stores[n_ch - 1] = pltpu.async_copy(
            rb[(n_ch - 1) % nbuf],
            out_hbm.at[pl.ds(base + (n_ch - 1) * chunk, chunk)],
            ss[(n_ch - 1) % nbuf])
        for c in range(max(0, n_ch - nbuf), n_ch):
            stores[c].wait()

    return k(table, idx2)


def _sc_scatter_sorted(qkv, bwd_idx):
    chunk = 128
    n_ch = _S // chunk
    n_units = _N * n_ch // _NW
    mesh = plsc.VectorSubcoreMesh(core_axis_name="c", subcore_axis_name="s")

    @functools.partial(
        pl.kernel, mesh=mesh,
        out_type=jax.ShapeDtypeStruct((_G * _S, 2 * _DH), jnp.float32),
        scratch_types=[pltpu.VMEM((n_units * _NHASH, chunk), jnp.int32),
                       pltpu.VMEM((chunk, 2 * _DH), jnp.float32),
                       pltpu.VMEM((chunk, 2 * _DH), jnp.float32),
                       pltpu.SemaphoreType.DMA,
                       pltpu.SemaphoreType.DMA,
                       pltpu.SemaphoreType.DMA,
                       pltpu.SemaphoreType.DMA,
                       pltpu.SemaphoreType.DMA],
    )
    def k(qkv_hbm, ridx_hbm, sort_hbm, idx_all, rb0, rb1,
          isem, ls0, ls1, ss0, ss1):
        rb = (rb0, rb1)
        ls = (ls0, ls1)
        ss = (ss0, ss1)
        wid = lax.axis_index("s") * 2 + lax.axis_index("c")

        icps = []
        for u in range(n_units):
            unit = wid * n_units + u
            n = unit // n_ch
            c = unit % n_ch
            for h in range(_NHASH):
                icps.append(pltpu.async_copy(
                    ridx_hbm.at[pl.ds((n * _NHASH + h) * _S + c * chunk, chunk)],
                    idx_all.at[u * _NHASH + h], isem))
        for cp in icps:
            cp.wait()

        loads = [None] * n_units
        scats = [None] * n_units
        for u in range(n_units):
            bi = u % 2
            if u >= 2:
                for cp in scats[u - 2]:
                    cp.wait()
            unit = wid * n_units + u
            n = unit // n_ch
            c = unit % n_ch
            loads[u] = pltpu.async_copy(
                qkv_hbm.at[pl.ds(n * _S + c * chunk, chunk)], rb[bi], ls[bi])
            if u >= 1:
                loads[u - 1].wait()
                scats[u - 1] = [
                    pltpu.async_copy(rb[(u - 1) % 2],
                                     sort_hbm.at[idx_all.at[(u - 1) * _NHASH + h]],
                                     ss[(u - 1) % 2])
                    for h in range(_NHASH)]
        loads[n_units - 1].wait()
        scats[n_units - 1] = [
            pltpu.async_copy(rb[(n_units - 1) % 2],
                             sort_hbm.at[idx_all.at[(n_units - 1) * _NHASH + h]],
                             ss[(n_units - 1) % 2])
            for h in range(_NHASH)]
        for u in (n_units - 2, n_units - 1):
            for cp in scats[u]:
                cp.wait()

    return k(qkv, bwd_idx)


_SB = 256
_NSB = _B * _S // _SB


def _ln_block(x, g, b):
    m = jnp.mean(x, axis=1, keepdims=True)
    v = jnp.mean((x - m) ** 2, axis=1, keepdims=True)
    return (x - m) / jnp.sqrt(v + 1e-5) * g + b


def _t_embed_ln(emb_rows, g, b):
    pe = jnp.asarray(_PE)

    def body(e_ref, p_ref, g_ref, b_ref, o_ref, oln_ref):
        x = e_ref[...] + p_ref[...]
        o_ref[...] = x
        oln_ref[...] = _ln_block(x, g_ref[...], b_ref[...])

    return pl.pallas_call(
        body,
        grid=(_NSB,),
        in_specs=[
            pl.BlockSpec((_SB, _D), lambda i: (i, 0)),
            pl.BlockSpec((_SB, _D), lambda i: (i % (_S // _SB), 0)),
            pl.BlockSpec((1, _D), lambda i: (0, 0)),
            pl.BlockSpec((1, _D), lambda i: (0, 0)),
        ],
        out_specs=[pl.BlockSpec((_SB, _D), lambda i: (i, 0)),
                   pl.BlockSpec((_SB, _D), lambda i: (i, 0))],
        out_shape=[jax.ShapeDtypeStruct((_B * _S, _D), jnp.float32),
                   jax.ShapeDtypeStruct((_B * _S, _D), jnp.float32)],
    )(emb_rows, pe, g.reshape(1, _D), b.reshape(1, _D))


def _t_qkv(xln, src_col, wqk_t, wv_t, r_flat):
    nb = _S // _SB

    def body(x_ref, s_ref, wq_ref, wv_ref, r_ref, o_ref, rot_ref):
        x = x_ref[0]
        pad = s_ref[0] == 0
        for hd in range(_H):
            wq = wq_ref[hd * _DH:(hd + 1) * _DH, :]
            wv = wv_ref[hd * _DH:(hd + 1) * _DH, :]
            qk = lax.dot_general(x, wq, (((1,), (1,)), ((), ())),
                                 preferred_element_type=jnp.float32)
            v = lax.dot_general(x, wv, (((1,), (1,)), ((), ())),
                                preferred_element_type=jnp.float32)
            rot = jnp.dot(qk, r_ref[...], preferred_element_type=jnp.float32)
            pm = []
            for h in range(_NHASH):
                rh = rot[:, h * 16:(h + 1) * 16]
                pm.extend([rh, -rh])
            rot_ref[0, hd] = jnp.concatenate(pm, axis=1)
            qkz = jnp.where(pad, 0.0, qk)
            vz = jnp.where(pad, 0.0, v)
            o_ref[0, hd] = jnp.concatenate([qkz, vz], axis=1)

    return pl.pallas_call(
        body,
        grid=(_NSB,),
        in_specs=[
            pl.BlockSpec((1, _SB, _D), lambda m: (m, 0, 0)),
            pl.BlockSpec((1, _SB, 1), lambda m: (m, 0, 0)),
            pl.BlockSpec((_D, _D), lambda m: (0, 0)),
            pl.BlockSpec((_D, _D), lambda m: (0, 0)),
            pl.BlockSpec((_DH, _DH), lambda m: (0, 0)),
        ],
        out_specs=[
            pl.BlockSpec((1, _H, _SB, 2 * _DH),
                         lambda m: (m // nb, 0, m % nb, 0)),
            pl.BlockSpec((1, _H, _SB, _NHASH * _NBKT),
                         lambda m: (m // nb, 0, m % nb, 0)),
        ],
        out_shape=[
            jax.ShapeDtypeStruct((_B, _H, _S, 2 * _DH), jnp.float32),
            jax.ShapeDtypeStruct((_B, _H, _S, _NHASH * _NBKT), jnp.float32),
        ],
    )(xln.reshape(_NSB, _SB, _D), src_col, wqk_t, wv_t, r_flat)


def _t_rank(rot):
    W = _NHASH * _NBKT

    def body(rot_ref, o_ref):
        n = pl.program_id(0)
        pm = rot_ref[0]
        lane = lax.broadcasted_iota(jnp.int32, (_S, _NBKT), 1)
        ohs = []
        for h in range(_NHASH):
            pmh = pm[:, h * _NBKT:(h + 1) * _NBKT]
            mx = jnp.max(pmh, axis=1, keepdims=True)
            amc = jnp.min(jnp.where(pmh == mx, lane, _NBKT + 1), axis=1,
                          keepdims=True)
            ohs.append((lane == amc).astype(jnp.float32))
        oh = jnp.concatenate(ohs, axis=1)
        inc = oh
        k = 1
        while k < _S:
            shifted = jnp.concatenate(
                [jnp.zeros((k, W), jnp.float32), inc[: _S - k]], axis=0)
            inc = inc + shifted
            k *= 2
        excl = inc - oh
        tot = inc[_S - 1:_S, :]
        l0 = lax.broadcasted_iota(jnp.int32, (W, W), 0)
        l1 = lax.broadcasted_iota(jnp.int32, (W, W), 1)
        ut = ((l0 // _NBKT == l1 // _NBKT) & (l0 < l1)).astype(jnp.float32)
        offs = jnp.dot(tot, ut, preferred_element_type=jnp.float32)
        prod = oh * (excl + offs)
        for h in range(_NHASH):
            rank = jnp.sum(prod[:, h * _NBKT:(h + 1) * _NBKT], axis=1,
                           keepdims=True)
            o_ref[0, h] = rank.astype(jnp.int32) + (n * _NHASH + h) * _S

    return pl.pallas_call(
        body,
        grid=(_N,),
        in_specs=[pl.BlockSpec((1, _S, W), lambda n: (n, 0, 0))],
        out_specs=pl.BlockSpec((1, _NHASH, _S, 1), lambda n: (n, 0, 0, 0)),
        out_shape=jax.ShapeDtypeStruct((_N, _NHASH, _S, 1), jnp.int32),
    )(rot)


def _t_attention(sorted_rows):
    scale = 1.0 / math.sqrt(_DH)

    QW = 2 * _BKT
    KW = 4 * _BKT

    def body(rows_ref, o_ref):
        r_i = lax.broadcasted_iota(jnp.int32, (QW, KW), 0)
        j_i = lax.broadcasted_iota(jnp.int32, (QW, KW), 1)
        self_mask = j_i == r_i + _BKT
        rq = r_i // _BKT
        rk = j_i // _BKT
        in_window = (rk == rq) | (rk == rq + 1)
        bias = jnp.where(self_mask, -1e5,
                         jnp.where(in_window, 0.0, -1e30))
        for t in range(_NCHUNK // 2):
            p = 2 * t
            cs = p * _BKT
            if t == 0:
                win = jnp.concatenate(
                    [rows_ref[0, (_S - _BKT):_S, :], rows_ref[0, 0:3 * _BKT, :]],
                    axis=0)
            elif t == _NCHUNK // 2 - 1:
                win = jnp.concatenate(
                    [rows_ref[0, cs - _BKT:_S, :], rows_ref[0, 0:_BKT, :]],
                    axis=0)
            else:
                win = rows_ref[0, cs - _BKT:cs + 3 * _BKT, :]
            q = rows_ref[0, cs:cs + QW, 0:_DH]
            kqk = win[:, 0:_DH]
            vv = win[:, _DH:2 * _DH]
            nrm = jnp.sqrt(jnp.sum(kqk * kqk, axis=1, keepdims=True))
            kn = kqk / (nrm + 1e-9)
            q_aug = jnp.concatenate(
                [q, jnp.ones((QW, 1), jnp.float32)], axis=1)
            kn_aug = jnp.concatenate(
                [kn, jnp.where(nrm == 0.0, -1e9 / scale, 0.0)], axis=1)
            dots = lax.dot_general(q_aug, kn_aug, (((1,), (1,)), ((), ())),
                                   preferred_element_type=jnp.float32) * scale
            dots = dots + bias
            mx = jnp.max(dots, axis=1, keepdims=True)
            e = jnp.exp(dots - mx)
            ssum = jnp.sum(e, axis=1, keepdims=True)
            lse = mx + jnp.log(ssum)
            o = jnp.dot(e, vv, preferred_element_type=jnp.float32) / ssum
            o_ref[0, cs:cs + QW, 0:_DH] = o
            o_ref[0, cs:cs + QW, _DH:_DH + 1] = lse

    return pl.pallas_call(
        body,
        grid=(_G,),
        in_specs=[pl.BlockSpec((1, _S, 2 * _DH), lambda g: (g, 0, 0))],
        out_specs=pl.BlockSpec((1, _S, 2 * _DH), lambda g: (g, 0, 0)),
        out_shape=jax.ShapeDtypeStruct((_G, _S, 2 * _DH), jnp.float32),
    )(sorted_rows)


def _t_combine(o_unsorted, x_res, wo, g, b):
    nb = _S // _SB

    def body(o_ref, x_ref, wo_ref, g_ref, b_ref, out_ref, oln_ref):
        parts = []
        for hd in range(_H):
            ls = [o_ref[0, hd * _NHASH + j, :, _DH:_DH + 1]
                  for j in range(_NHASH)]
            os_ = [o_ref[0, hd * _NHASH + j, :, 0:_DH] for j in range(_NHASH)]
            mx = jnp.maximum(jnp.maximum(ls[0], ls[1]),
                             jnp.maximum(ls[2], ls[3]))
            ws = [jnp.exp(l - mx) for l in ls]
            tot = ws[0] + ws[1] + ws[2] + ws[3]
            ctx = (ws[0] * os_[0] + ws[1] * os_[1]
                   + ws[2] * os_[2] + ws[3] * os_[3]) / tot
            parts.append(ctx)
        ctx = jnp.concatenate(parts, axis=1)
        x1 = x_ref[0] + jnp.dot(ctx, wo_ref[...],
                                preferred_element_type=jnp.float32)
        out_ref[0] = x1
        oln_ref[0] = _ln_block(x1, g_ref[...], b_ref[...])

    return pl.pallas_call(
        body,
        grid=(_B, nb),
        in_specs=[
            pl.BlockSpec((1, _H * _NHASH, _SB, 2 * _DH),
                         lambda b, s: (b, 0, s, 0)),
            pl.BlockSpec((1, _SB, _D), lambda b, s: (b * nb + s, 0, 0)),
            pl.BlockSpec((_D, _D), lambda b, s: (0, 0)),
            pl.BlockSpec((1, _D), lambda b, s: (0, 0)),
            pl.BlockSpec((1, _D), lambda b, s: (0, 0)),
        ],
        out_specs=[
            pl.BlockSpec((1, _SB, _D), lambda b, s: (b * nb + s, 0, 0)),
            pl.BlockSpec((1, _SB, _D), lambda b, s: (b * nb + s, 0, 0)),
        ],
        out_shape=[jax.ShapeDtypeStruct((_NSB, _SB, _D), jnp.float32),
                   jax.ShapeDtypeStruct((_NSB, _SB, _D), jnp.float32)],
    )(o_unsorted.reshape(_B, _H * _NHASH, _S, 2 * _DH),
      x_res.reshape(_NSB, _SB, _D), wo, g.reshape(1, _D), b.reshape(1, _D))


def _t_ffn1(xln, w1, b1):
    def body(x_ref, w_ref, b1_ref, o_ref):
        o_ref[0] = jax.nn.gelu(
            jnp.dot(x_ref[0], w_ref[...], preferred_element_type=jnp.float32)
            + b1_ref[...])

    return pl.pallas_call(
        body,
        grid=(_NSB,),
        in_specs=[
            pl.BlockSpec((1, _SB, _D), lambda m: (m, 0, 0)),
            pl.BlockSpec((_D, 4 * _D), lambda m: (0, 0)),
            pl.BlockSpec((1, 4 * _D), lambda m: (0, 0)),
        ],
        out_specs=pl.BlockSpec((1, _SB, 4 * _D), lambda m: (m, 0, 0)),
        out_shape=jax.ShapeDtypeStruct((_NSB, _SB, 4 * _D), jnp.float32),
    )(xln, w1, b1.reshape(1, 4 * _D))


def _t_ffn2(h1, x_res, w2, b2, lng=None, lnb=None):
    with_ln = lng is not None

    def body(h_ref, x_ref, w_ref, b_ref, *rest):
        x = (x_ref[0]
             + jnp.dot(h_ref[0], w_ref[...], preferred_element_type=jnp.float32)
             + b_ref[...])
        if with_ln:
            g_ref, lb_ref, o_ref, oln_ref = rest
            o_ref[0] = x
            oln_ref[0] = _ln_block(x, g_ref[...], lb_ref[...])
        else:
            (o_ref,) = rest
            o_ref[0] = x

    in_specs = [
        pl.BlockSpec((1, _SB, 4 * _D), lambda m: (m, 0, 0)),
        pl.BlockSpec((1, _SB, _D), lambda m: (m, 0, 0)),
        pl.BlockSpec((4 * _D, _D), lambda m: (0, 0)),
        pl.BlockSpec((1, _D), lambda m: (0, 0)),
    ]
    args = [h1, x_res.reshape(_NSB, _SB, _D), w2, b2.reshape(1, _D)]
    out_spec = pl.BlockSpec((1, _SB, _D), lambda m: (m, 0, 0))
    if with_ln:
        in_specs += [pl.BlockSpec((1, _D), lambda m: (0, 0))] * 2
        args += [lng.reshape(1, _D), lnb.reshape(1, _D)]
        out_specs = [out_spec, out_spec]
        out_shape = [jax.ShapeDtypeStruct((_NSB, _SB, _D), jnp.float32)] * 2
    else:
        out_specs = out_spec
        out_shape = jax.ShapeDtypeStruct((_NSB, _SB, _D), jnp.float32)

    return pl.pallas_call(
        body,
        grid=(_NSB,),
        in_specs=in_specs,
        out_specs=out_specs,
        out_shape=out_shape,
    )(*args)


def _t_classifier(x, src_col, wp, bp, wc_pad, bc_pad):
    def body(x_ref, s_ref, wp_ref, bp_ref, wc_ref, bc_ref, o_ref):
        keep = (s_ref[0] != 0).astype(jnp.float32)
        hidden = x_ref[0] * keep
        summed = jnp.sum(hidden, axis=0, keepdims=True)
        cnt = jnp.sum(keep, axis=0, keepdims=True)
        pooled = summed / cnt
        pr = jnp.maximum(
            jnp.dot(pooled, wp_ref[...], preferred_element_type=jnp.float32)
            + bp_ref[...], 0.0)
        o_ref[0] = (jnp.dot(pr, wc_ref[...], preferred_element_type=jnp.float32)
                    + bc_ref[...])

    return pl.pallas_call(
        body,
        grid=(_B,),
        in_specs=[
            pl.BlockSpec((1, _S, _D), lambda b: (b, 0, 0)),
            pl.BlockSpec((1, _S, 1), lambda b: (b, 0, 0)),
            pl.BlockSpec((_D, _D), lambda b: (0, 0)),
            pl.BlockSpec((1, _D), lambda b: (0, 0)),
            pl.BlockSpec((_D, 64), lambda b: (0, 0)),
            pl.BlockSpec((1, 64), lambda b: (0, 0)),
        ],
        out_specs=pl.BlockSpec((1, 1, 64), lambda b: (b, 0, 0)),
        out_shape=jax.ShapeDtypeStruct((_B, 1, 64), jnp.float32),
    )(x.reshape(_B, _S, _D), src_col, wp, bp.reshape(1, _D), wc_pad, bc_pad)


def _layer(x, xln, p, src_col, next_ln):
    wqk_t = p['Wqk'].T
    wv_t = p['Wv'].T
    r_flat = p['rotations'].reshape(_DH, _DH)
    qkv, rot = _t_qkv(xln, src_col, wqk_t, wv_t, r_flat)
    bwd_idx = _t_rank(rot.reshape(_N, _S, _NHASH * _NBKT))
    bwd_flat = bwd_idx.reshape(_G * _S)
    sorted_rows = _sc_scatter_sorted(qkv.reshape(_N * _S, 2 * _DH), bwd_flat)
    so = _t_attention(sorted_rows.reshape(_G, _S, 2 * _DH))
    o_uns = _sc_gather_rows(so.reshape(_G * _S, 2 * _DH), bwd_flat)
    x1, xln2 = _t_combine(o_uns, x, p['Wo'], p['ln2_g'], p['ln2_b'])
    x1 = x1.reshape(_B * _S, _D)
    h1 = _t_ffn1(xln2, p['W1'], p['b1f'])
    if next_ln is None:
        x2 = _t_ffn2(h1, x1, p['W2'], p['b2f'])
        return x2.reshape(_B * _S, _D), None
    x2, xlnn = _t_ffn2(h1, x1, p['W2'], p['b2f'], next_ln[0], next_ln[1])
    return x2.reshape(_B * _S, _D), xlnn


def kernel(src, source_lengths, params):
    del source_lengths
    src = src.astype(jnp.int32)
    emb_rows = _sc_gather_rows(params['emb'], src.reshape(_B * _S))
    layers = params['layers']
    x, xln = _t_embed_ln(emb_rows, layers[0]['ln1_g'], layers[0]['ln1_b'])
    src_col = src.reshape(_NSB, _SB, 1)
    for li, p in enumerate(layers):
        nxt = None
        if li + 1 < len(layers):
            nxt = (layers[li + 1]['ln1_g'], layers[li + 1]['ln1_b'])
        x, xln = _layer(x, xln, p, src_col, nxt)
    wc_pad = jnp.pad(params['Wc'], ((0, 0), (0, 64 - _NCLS)))
    bc_pad = jnp.pad(params['bc'], (0, 64 - _NCLS)).reshape(1, 64)
    logits = _t_classifier(x, src.reshape(_B, _S, 1), params['Wp'],
                           params['bp'], wc_pad, bc_pad)
    return logits.reshape(_B, 64)[:, :_NCLS]


# speedup vs baseline: 1.0442x; 1.0442x over previous
import functools
import math

import numpy as np
import jax
import jax.numpy as jnp
from jax import lax
from jax.experimental import pallas as pl
from jax.experimental.pallas import tpu as pltpu
from jax.experimental.pallas import tpu_sc as plsc

_VOCAB = 50000
_D = 768
_H = 12
_DH = 64
_S = 2048
_B = 2
_NHASH = 4
_BKT = 64
_NBKT = 32
_NCLS = 50
_N = _B * _H
_G = _N * _NHASH
_NCHUNK = _S // _BKT
_NW = 32


def _pe_table():
    pos = np.arange(_S)[:, None].astype(np.float32)
    div = np.exp(np.arange(0, _D, 2).astype(np.float32) * (-np.log(10000.0) / _D))
    pe = np.zeros((_S, _D), dtype=np.float32)
    pe[:, 0::2] = np.sin(pos * div)
    pe[:, 1::2] = np.cos(pos * div)
    return pe

_PE = _pe_table()


def _sc_gather_rows(table, idx, chunk=128):
    nrow, d = table.shape
    (ng,) = idx.shape
    per_w = ng // _NW
    chunk = min(chunk, per_w)
    n_ch = per_w // chunk
    assert per_w % chunk == 0 and ng % _NW == 0
    nbuf = min(3, n_ch)
    idx2 = idx.reshape(_NW * n_ch, chunk)
    mesh = plsc.VectorSubcoreMesh(core_axis_name="c", subcore_axis_name="s")

    @functools.partial(
        pl.kernel, mesh=mesh,
        out_type=jax.ShapeDtypeStruct((ng, d), jnp.float32),
        scratch_types=(
            [pltpu.VMEM((n_ch, chunk), jnp.int32)]
            + [pltpu.VMEM((chunk, d), jnp.float32)] * nbuf
            + [pltpu.SemaphoreType.DMA] * (2 * nbuf)),
    )
    def k(table_hbm, idx_hbm, out_hbm, idx_all, *bufs_sems):
        rb = bufs_sems[:nbuf]
        gs = bufs_sems[nbuf:2 * nbuf]
        ss = bufs_sems[2 * nbuf:]
        wid = lax.axis_index("s") * 2 + lax.axis_index("c")
        base = wid * per_w
        pltpu.sync_copy(idx_hbm.at[pl.ds(wid * n_ch, n_ch)], idx_all)
        gathers = [None] * n_ch
        stores = [None] * n_ch
        for c in range(n_ch):
            bi = c % nbuf
            if c - nbuf >= 0:
                stores[c - nbuf].wait()
            gathers[c] = pltpu.async_copy(
                table_hbm.at[idx_all.at[c]], rb[bi], gs[bi])
            if c >= 1:
                gathers[c - 1].wait()
                stores[c - 1] = pltpu.async_copy(
                    rb[(c - 1) % nbuf],
                    out_hbm.at[pl.ds(base + (c - 1) * chunk, chunk)],
                    ss[(c - 1) % nbuf])
        gathers[n_ch - 1].wait()
        stores[n_ch - 1] = pltpu.async_copy(
            rb[(n_ch - 1) % nbuf],
            out_hbm.at[pl.ds(base + (n_ch - 1) * chunk, chunk)],
            ss[(n_ch - 1) % nbuf])
        for c in range(max(0, n_ch - nbuf), n_ch):
            stores[c].wait()

    return k(table, idx2)


def _sc_scatter_sorted(qkv, bwd_idx, n0, n_half):
    chunk = 128
    n_ch = _S // chunk
    n_units = n_half * n_ch // _NW
    mesh = plsc.VectorSubcoreMesh(core_axis_name="c", subcore_axis_name="s")

    @functools.partial(
        pl.kernel, mesh=mesh,
        out_type=jax.ShapeDtypeStruct((n_half * _NHASH * _S, 2 * _DH),
                                      jnp.float32),
        scratch_types=[pltpu.VMEM((n_units * _NHASH, chunk), jnp.int32),
                       pltpu.VMEM((chunk, 2 * _DH), jnp.float32),
                       pltpu.VMEM((chunk, 2 * _DH), jnp.float32),
                       pltpu.SemaphoreType.DMA,
                       pltpu.SemaphoreType.DMA,
                       pltpu.SemaphoreType.DMA,
                       pltpu.SemaphoreType.DMA,
                       pltpu.SemaphoreType.DMA],
    )
    def k(qkv_hbm, ridx_hbm, sort_hbm, idx_all, rb0, rb1,
          isem, ls0, ls1, ss0, ss1):
        rb = (rb0, rb1)
        ls = (ls0, ls1)
        ss = (ss0, ss1)
        wid = lax.axis_index("s") * 2 + lax.axis_index("c")

        icps = []
        for u in range(n_units):
            unit = wid * n_units + u
            n = unit // n_ch
            c = unit % n_ch
            for h in range(_NHASH):
                icps.append(pltpu.async_copy(
                    ridx_hbm.at[pl.ds((n * _NHASH + h) * _S + c * chunk, chunk)],
                    idx_all.at[u * _NHASH + h], isem))
        for cp in icps:
            cp.wait()

        loads = [None] * n_units
        scats = [None] * n_units
        for u in range(n_units):
            bi = u % 2
            if u >= 2:
                for cp in scats[u - 2]:
                    cp.wait()
            unit = wid * n_units + u
            n = unit // n_ch
            c = unit % n_ch
            loads[u] = pltpu.async_copy(
                qkv_hbm.at[pl.ds((n0 + n) * _S + c * chunk, chunk)],
                rb[bi], ls[bi])
            if u >= 1:
                loads[u - 1].wait()
                scats[u - 1] = [
                    pltpu.async_copy(rb[(u - 1) % 2],
                                     sort_hbm.at[idx_all.at[(u - 1) * _NHASH + h]],
                                     ss[(u - 1) % 2])
                    for h in range(_NHASH)]
        loads[n_units - 1].wait()
        scats[n_units - 1] = [
            pltpu.async_copy(rb[(n_units - 1) % 2],
                             sort_hbm.at[idx_all.at[(n_units - 1) * _NHASH + h]],
                             ss[(n_units - 1) % 2])
            for h in range(_NHASH)]
        for u in (n_units - 2, n_units - 1):
            for cp in scats[u]:
                cp.wait()

    return k(qkv, bwd_idx)


_NHALF = _N // _B


_SB = 256
_NSB = _B * _S // _SB


def _ln_block(x, g, b):
    m = jnp.mean(x, axis=1, keepdims=True)
    v = jnp.mean((x - m) ** 2, axis=1, keepdims=True)
    return (x - m) / jnp.sqrt(v + 1e-5) * g + b


def _t_embed_ln(emb_rows, g, b):
    pe = jnp.asarray(_PE)

    def body(e_ref, p_ref, g_ref, b_ref, o_ref, oln_ref):
        x = e_ref[...] + p_ref[...]
        o_ref[...] = x
        oln_ref[...] = _ln_block(x, g_ref[...], b_ref[...])

    return pl.pallas_call(
        body,
        grid=(_NSB,),
        in_specs=[
            pl.BlockSpec((_SB, _D), lambda i: (i, 0)),
            pl.BlockSpec((_SB, _D), lambda i: (i % (_S // _SB), 0)),
            pl.BlockSpec((1, _D), lambda i: (0, 0)),
            pl.BlockSpec((1, _D), lambda i: (0, 0)),
        ],
        out_specs=[pl.BlockSpec((_SB, _D), lambda i: (i, 0)),
                   pl.BlockSpec((_SB, _D), lambda i: (i, 0))],
        out_shape=[jax.ShapeDtypeStruct((_B * _S, _D), jnp.float32),
                   jax.ShapeDtypeStruct((_B * _S, _D), jnp.float32)],
    )(emb_rows, pe, g.reshape(1, _D), b.reshape(1, _D))


def _t_qkv(xln, src_col, wqk_t, wv_t, r_flat):
    nb = _S // _SB

    def body(x_ref, s_ref, wq_ref, wv_ref, r_ref, o_ref, rot_ref):
        x = x_ref[0]
        pad = s_ref[0] == 0
        for hd in range(_H):
            wq = wq_ref[hd * _DH:(hd + 1) * _DH, :]
            wv = wv_ref[hd * _DH:(hd + 1) * _DH, :]
            qk = lax.dot_general(x, wq, (((1,), (1,)), ((), ())),
                                 preferred_element_type=jnp.float32)
            v = lax.dot_general(x, wv, (((1,), (1,)), ((), ())),
                                preferred_element_type=jnp.float32)
            rot = jnp.dot(qk, r_ref[...], preferred_element_type=jnp.float32)
            pm = []
            for h in range(_NHASH):
                rh = rot[:, h * 16:(h + 1) * 16]
                pm.extend([rh, -rh])
            rot_ref[0, hd] = jnp.concatenate(pm, axis=1)
            qkz = jnp.where(pad, 0.0, qk)
            vz = jnp.where(pad, 0.0, v)
            o_ref[0, hd] = jnp.concatenate([qkz, vz], axis=1)

    return pl.pallas_call(
        body,
        grid=(_NSB,),
        in_specs=[
            pl.BlockSpec((1, _SB, _D), lambda m: (m, 0, 0)),
            pl.BlockSpec((1, _SB, 1), lambda m: (m, 0, 0)),
            pl.BlockSpec((_D, _D), lambda m: (0, 0)),
            pl.BlockSpec((_D, _D), lambda m: (0, 0)),
            pl.BlockSpec((_DH, _DH), lambda m: (0, 0)),
        ],
        out_specs=[
            pl.BlockSpec((1, _H, _SB, 2 * _DH),
                         lambda m: (m // nb, 0, m % nb, 0)),
            pl.BlockSpec((1, _H, _SB, _NHASH * _NBKT),
                         lambda m: (m // nb, 0, m % nb, 0)),
        ],
        out_shape=[
            jax.ShapeDtypeStruct((_B, _H, _S, 2 * _DH), jnp.float32),
            jax.ShapeDtypeStruct((_B, _H, _S, _NHASH * _NBKT), jnp.float32),
        ],
    )(xln.reshape(_NSB, _SB, _D), src_col, wqk_t, wv_t, r_flat)


def _t_rank(rot):
    W = _NHASH * _NBKT

    def body(rot_ref, o_ref):
        n = pl.program_id(0)
        pm = rot_ref[0]
        lane = lax.broadcasted_iota(jnp.int32, (_S, _NBKT), 1)
        ohs = []
        for h in range(_NHASH):
            pmh = pm[:, h * _NBKT:(h + 1) * _NBKT]
            mx = jnp.max(pmh, axis=1, keepdims=True)
            amc = jnp.min(jnp.where(pmh == mx, lane, _NBKT + 1), axis=1,
                          keepdims=True)
            ohs.append((lane == amc).astype(jnp.float32))
        oh = jnp.concatenate(ohs, axis=1)
        inc = oh
        k = 1
        while k < _S:
            shifted = jnp.concatenate(
                [jnp.zeros((k, W), jnp.float32), inc[: _S - k]], axis=0)
            inc = inc + shifted
            k *= 2
        excl = inc - oh
        tot = inc[_S - 1:_S, :]
        l0 = lax.broadcasted_iota(jnp.int32, (W, W), 0)
        l1 = lax.broadcasted_iota(jnp.int32, (W, W), 1)
        ut = ((l0 // _NBKT == l1 // _NBKT) & (l0 < l1)).astype(jnp.float32)
        offs = jnp.dot(tot, ut, preferred_element_type=jnp.float32)
        prod = oh * (excl + offs)
        n_loc = n % _NHALF
        for h in range(_NHASH):
            rank = jnp.sum(prod[:, h * _NBKT:(h + 1) * _NBKT], axis=1,
                           keepdims=True)
            o_ref[0, h] = rank.astype(jnp.int32) + (n_loc * _NHASH + h) * _S

    return pl.pallas_call(
        body,
        grid=(_N,),
        in_specs=[pl.BlockSpec((1, _S, W), lambda n: (n, 0, 0))],
        out_specs=pl.BlockSpec((1, _NHASH, _S, 1), lambda n: (n, 0, 0, 0)),
        out_shape=jax.ShapeDtypeStruct((_N, _NHASH, _S, 1), jnp.int32),
    )(rot)


def _t_attention(sorted_rows):
    scale = 1.0 / math.sqrt(_DH)

    QW = 2 * _BKT
    KW = 4 * _BKT

    def body(rows_ref, o_ref):
        r_i = lax.broadcasted_iota(jnp.int32, (QW, KW), 0)
        j_i = lax.broadcasted_iota(jnp.int32, (QW, KW), 1)
        self_mask = j_i == r_i + _BKT
        rq = r_i // _BKT
        rk = j_i // _BKT
        in_window = (rk == rq) | (rk == rq + 1)
        bias = jnp.where(self_mask, -1e5,
                         jnp.where(in_window, 0.0, -1e30))
        for t in range(_NCHUNK // 2):
            p = 2 * t
            cs = p * _BKT
            if t == 0:
                win = jnp.concatenate(
                    [rows_ref[0, (_S - _BKT):_S, :], rows_ref[0, 0:3 * _BKT, :]],
                    axis=0)
            elif t == _NCHUNK // 2 - 1:
                win = jnp.concatenate(
                    [rows_ref[0, cs - _BKT:_S, :], rows_ref[0, 0:_BKT, :]],
                    axis=0)
            else:
                win = rows_ref[0, cs - _BKT:cs + 3 * _BKT, :]
            q = rows_ref[0, cs:cs + QW, 0:_DH]
            kqk = win[:, 0:_DH]
            vv = win[:, _DH:2 * _DH]
            nrm = jnp.sqrt(jnp.sum(kqk * kqk, axis=1, keepdims=True))
            kn = kqk / (nrm + 1e-9)
            dots = lax.dot_general(q, kn, (((1,), (1,)), ((), ())),
                                   preferred_element_type=jnp.float32) * scale
            padk = jnp.sum(jnp.abs(dots), axis=0, keepdims=True) == 0.0
            dots = jnp.where(padk, -1e9, dots + bias)
            mx = jnp.max(dots, axis=1, keepdims=True)
            e = jnp.exp(dots - mx)
            ssum = jnp.sum(e, axis=1, keepdims=True)
            lse = mx + jnp.log(ssum)
            o = jnp.dot(e, vv, preferred_element_type=jnp.float32) / ssum
            o_ref[0, cs:cs + QW, 0:_DH] = o
            o_ref[0, cs:cs + QW, _DH:_DH + 1] = lse

    ng = sorted_rows.shape[0]
    return pl.pallas_call(
        body,
        grid=(ng,),
        in_specs=[pl.BlockSpec((1, _S, 2 * _DH), lambda g: (g, 0, 0))],
        out_specs=pl.BlockSpec((1, _S, 2 * _DH), lambda g: (g, 0, 0)),
        out_shape=jax.ShapeDtypeStruct((ng, _S, 2 * _DH), jnp.float32),
    )(sorted_rows)


def _t_combine(o_unsorted, x_res_b, wo, g, b):
    nb = _S // _SB

    def body(o_ref, x_ref, wo_ref, g_ref, b_ref, out_ref, oln_ref):
        parts = []
        for hd in range(_H):
            ls = [o_ref[hd * _NHASH + j, :, _DH:_DH + 1]
                  for j in range(_NHASH)]
            os_ = [o_ref[hd * _NHASH + j, :, 0:_DH] for j in range(_NHASH)]
            mx = jnp.maximum(jnp.maximum(ls[0], ls[1]),
                             jnp.maximum(ls[2], ls[3]))
            ws = [jnp.exp(l - mx) for l in ls]
            tot = ws[0] + ws[1] + ws[2] + ws[3]
            ctx = (ws[0] * os_[0] + ws[1] * os_[1]
                   + ws[2] * os_[2] + ws[3] * os_[3]) / tot
            parts.append(ctx)
        ctx = jnp.concatenate(parts, axis=1)
        x1 = x_ref[0] + jnp.dot(ctx, wo_ref[...],
                                preferred_element_type=jnp.float32)
        out_ref[0] = x1
        oln_ref[0] = _ln_block(x1, g_ref[...], b_ref[...])

    return pl.pallas_call(
        body,
        grid=(nb,),
        in_specs=[
            pl.BlockSpec((_H * _NHASH, _SB, 2 * _DH), lambda s: (0, s, 0)),
            pl.BlockSpec((1, _SB, _D), lambda s: (s, 0, 0)),
            pl.BlockSpec((_D, _D), lambda s: (0, 0)),
            pl.BlockSpec((1, _D), lambda s: (0, 0)),
            pl.BlockSpec((1, _D), lambda s: (0, 0)),
        ],
        out_specs=[
            pl.BlockSpec((1, _SB, _D), lambda s: (s, 0, 0)),
            pl.BlockSpec((1, _SB, _D), lambda s: (s, 0, 0)),
        ],
        out_shape=[jax.ShapeDtypeStruct((nb, _SB, _D), jnp.float32),
                   jax.ShapeDtypeStruct((nb, _SB, _D), jnp.float32)],
    )(o_unsorted.reshape(_H * _NHASH, _S, 2 * _DH),
      x_res_b.reshape(nb, _SB, _D), wo, g.reshape(1, _D), b.reshape(1, _D))


def _t_ffn1(xln, w1, b1):
    def body(x_ref, w_ref, b1_ref, o_ref):
        o_ref[0] = jax.nn.gelu(
            jnp.dot(x_ref[0], w_ref[...], preferred_element_type=jnp.float32)
            + b1_ref[...])

    return pl.pallas_call(
        body,
        grid=(_NSB,),
        in_specs=[
            pl.BlockSpec((1, _SB, _D), lambda m: (m, 0, 0)),
            pl.BlockSpec((_D, 4 * _D), lambda m: (0, 0)),
            pl.BlockSpec((1, 4 * _D), lambda m: (0, 0)),
        ],
        out_specs=pl.BlockSpec((1, _SB, 4 * _D), lambda m: (m, 0, 0)),
        out_shape=jax.ShapeDtypeStruct((_NSB, _SB, 4 * _D), jnp.float32),
    )(xln, w1, b1.reshape(1, 4 * _D))


def _t_ffn2(h1, x_res, w2, b2, lng=None, lnb=None):
    with_ln = lng is not None

    def body(h_ref, x_ref, w_ref, b_ref, *rest):
        x = (x_ref[0]
             + jnp.dot(h_ref[0], w_ref[...], preferred_element_type=jnp.float32)
             + b_ref[...])
        if with_ln:
            g_ref, lb_ref, o_ref, oln_ref = rest
            o_ref[0] = x
            oln_ref[0] = _ln_block(x, g_ref[...], lb_ref[...])
        else:
            (o_ref,) = rest
            o_ref[0] = x

    in_specs = [
        pl.BlockSpec((1, _SB, 4 * _D), lambda m: (m, 0, 0)),
        pl.BlockSpec((1, _SB, _D), lambda m: (m, 0, 0)),
        pl.BlockSpec((4 * _D, _D), lambda m: (0, 0)),
        pl.BlockSpec((1, _D), lambda m: (0, 0)),
    ]
    args = [h1, x_res.reshape(_NSB, _SB, _D), w2, b2.reshape(1, _D)]
    out_spec = pl.BlockSpec((1, _SB, _D), lambda m: (m, 0, 0))
    if with_ln:
        in_specs += [pl.BlockSpec((1, _D), lambda m: (0, 0))] * 2
        args += [lng.reshape(1, _D), lnb.reshape(1, _D)]
        out_specs = [out_spec, out_spec]
        out_shape = [jax.ShapeDtypeStruct((_NSB, _SB, _D), jnp.float32)] * 2
    else:
        out_specs = out_spec
        out_shape = jax.ShapeDtypeStruct((_NSB, _SB, _D), jnp.float32)

    return pl.pallas_call(
        body,
        grid=(_NSB,),
        in_specs=in_specs,
        out_specs=out_specs,
        out_shape=out_shape,
    )(*args)


def _t_classifier(x, src_col, wp, bp, wc_pad, bc_pad):
    def body(x_ref, s_ref, wp_ref, bp_ref, wc_ref, bc_ref, o_ref):
        keep = (s_ref[0] != 0).astype(jnp.float32)
        hidden = x_ref[0] * keep
        summed = jnp.sum(hidden, axis=0, keepdims=True)
        cnt = jnp.sum(keep, axis=0, keepdims=True)
        pooled = summed / cnt
        pr = jnp.maximum(
            jnp.dot(pooled, wp_ref[...], preferred_element_type=jnp.float32)
            + bp_ref[...], 0.0)
        o_ref[0] = (jnp.dot(pr, wc_ref[...], preferred_element_type=jnp.float32)
                    + bc_ref[...])

    return pl.pallas_call(
        body,
        grid=(_B,),
        in_specs=[
            pl.BlockSpec((1, _S, _D), lambda b: (b, 0, 0)),
            pl.BlockSpec((1, _S, 1), lambda b: (b, 0, 0)),
            pl.BlockSpec((_D, _D), lambda b: (0, 0)),
            pl.BlockSpec((1, _D), lambda b: (0, 0)),
            pl.BlockSpec((_D, 64), lambda b: (0, 0)),
            pl.BlockSpec((1, 64), lambda b: (0, 0)),
        ],
        out_specs=pl.BlockSpec((1, 1, 64), lambda b: (b, 0, 0)),
        out_shape=jax.ShapeDtypeStruct((_B, 1, 64), jnp.float32),
    )(x.reshape(_B, _S, _D), src_col, wp, bp.reshape(1, _D), wc_pad, bc_pad)


def _layer(x, xln, p, src_col, next_ln):
    wqk_t = p['Wqk'].T
    wv_t = p['Wv'].T
    r_flat = p['rotations'].reshape(_DH, _DH)
    qkv, rot = _t_qkv(xln, src_col, wqk_t, wv_t, r_flat)
    bwd_idx = _t_rank(rot.reshape(_N, _S, _NHASH * _NBKT))
    bwd_halves = bwd_idx.reshape(_B, _NHALF * _NHASH * _S)
    qkv_flat = qkv.reshape(_N * _S, 2 * _DH)
    gh = _NHALF * _NHASH
    x1s, xln2s = [], []
    for half in range(_B):
        bwd_h = bwd_halves[half]
        sorted_h = _sc_scatter_sorted(qkv_flat, bwd_h, half * _NHALF, _NHALF)
        so_h = _t_attention(sorted_h.reshape(gh, _S, 2 * _DH))
        ou_h = _sc_gather_rows(so_h.reshape(gh * _S, 2 * _DH), bwd_h)
        x1_h, xln2_h = _t_combine(ou_h, x.reshape(_B, _S, _D)[half],
                                  p['Wo'], p['ln2_g'], p['ln2_b'])
        x1s.append(x1_h.reshape(_S, _D))
        xln2s.append(xln2_h.reshape(_S, _D))
    x1 = jnp.concatenate(x1s, axis=0)
    xln2 = jnp.concatenate(xln2s, axis=0).reshape(_NSB, _SB, _D)
    h1 = _t_ffn1(xln2, p['W1'], p['b1f'])
    if next_ln is None:
        x2 = _t_ffn2(h1, x1, p['W2'], p['b2f'])
        return x2.reshape(_B * _S, _D), None
    x2, xlnn = _t_ffn2(h1, x1, p['W2'], p['b2f'], next_ln[0], next_ln[1])
    return x2.reshape(_B * _S, _D), xlnn


def kernel(src, source_lengths, params):
    del source_lengths
    src = src.astype(jnp.int32)
    emb_rows = _sc_gather_rows(params['emb'], src.reshape(_B * _S))
    layers = params['layers']
    x, xln = _t_embed_ln(emb_rows, layers[0]['ln1_g'], layers[0]['ln1_b'])
    src_col = src.reshape(_NSB, _SB, 1)
    for li, p in enumerate(layers):
        nxt = None
        if li + 1 < len(layers):
            nxt = (layers[li + 1]['ln1_g'], layers[li + 1]['ln1_b'])
        x, xln = _layer(x, xln, p, src_col, nxt)
    wc_pad = jnp.pad(params['Wc'], ((0, 0), (0, 64 - _NCLS)))
    bc_pad = jnp.pad(params['bc'], (0, 64 - _NCLS)).reshape(1, 64)
    logits = _t_classifier(x, src.reshape(_B, _S, 1), params['Wp'],
                           params['bp'], wc_pad, bc_pad)
    return logits.reshape(_B, 64)[:, :_NCLS]


# speedup vs baseline: 1.0477x; 1.0034x over previous
import functools
import math

import numpy as np
import jax
import jax.numpy as jnp
from jax import lax
from jax.experimental import pallas as pl
from jax.experimental.pallas import tpu as pltpu
from jax.experimental.pallas import tpu_sc as plsc

_VOCAB = 50000
_D = 768
_H = 12
_DH = 64
_S = 2048
_B = 2
_NHASH = 4
_BKT = 64
_NBKT = 32
_NCLS = 50
_N = _B * _H
_G = _N * _NHASH
_NCHUNK = _S // _BKT
_NW = 32


def _pe_table():
    pos = np.arange(_S)[:, None].astype(np.float32)
    div = np.exp(np.arange(0, _D, 2).astype(np.float32) * (-np.log(10000.0) / _D))
    pe = np.zeros((_S, _D), dtype=np.float32)
    pe[:, 0::2] = np.sin(pos * div)
    pe[:, 1::2] = np.cos(pos * div)
    return pe

_PE = _pe_table()


def _sc_gather_rows(table, idx, chunk=128):
    nrow, d = table.shape
    (ng,) = idx.shape
    per_w = ng // _NW
    chunk = min(chunk, per_w)
    n_ch = per_w // chunk
    assert per_w % chunk == 0 and ng % _NW == 0
    nbuf = min(4, n_ch)
    idx2 = idx.reshape(_NW * n_ch, chunk)
    mesh = plsc.VectorSubcoreMesh(core_axis_name="c", subcore_axis_name="s")

    @functools.partial(
        pl.kernel, mesh=mesh,
        out_type=jax.ShapeDtypeStruct((ng, d), jnp.float32),
        scratch_types=(
            [pltpu.VMEM((n_ch, chunk), jnp.int32)]
            + [pltpu.VMEM((chunk, d), jnp.float32)] * nbuf
            + [pltpu.SemaphoreType.DMA] * (2 * nbuf)),
    )
    def k(table_hbm, idx_hbm, out_hbm, idx_all, *bufs_sems):
        rb = bufs_sems[:nbuf]
        gs = bufs_sems[nbuf:2 * nbuf]
        ss = bufs_sems[2 * nbuf:]
        wid = lax.axis_index("s") * 2 + lax.axis_index("c")
        base = wid * per_w
        pltpu.sync_copy(idx_hbm.at[pl.ds(wid * n_ch, n_ch)], idx_all)
        gathers = [None] * n_ch
        stores = [None] * n_ch
        for c in range(n_ch):
            bi = c % nbuf
            if c - nbuf >= 0:
                stores[c - nbuf].wait()
            gathers[c] = pltpu.async_copy(
                table_hbm.at[idx_all.at[c]], rb[bi], gs[bi])
            if c >= 1:
                gathers[c - 1].wait()
                stores[c - 1] = pltpu.async_copy(
                    rb[(c - 1) % nbuf],
                    out_hbm.at[pl.ds(base + (c - 1) * chunk, chunk)],
                    ss[(c - 1) % nbuf])
        gathers[n_ch - 1].wait()
        stores[n_ch - 1] = pltpu.async_copy(
            rb[(n_ch - 1) % nbuf],
            out_hbm.at[pl.ds(base + (n_ch - 1) * chunk, chunk)],
            ss[(n_ch - 1) % nbuf])
        for c in range(max(0, n_ch - nbuf), n_ch):
            stores[c].wait()

    return k(table, idx2)


def _sc_scatter_sorted(qkv, bwd_idx, n0, n_half):
    chunk = 128
    n_ch = _S // chunk
    n_units = n_half * n_ch // _NW
    mesh = plsc.VectorSubcoreMesh(core_axis_name="c", subcore_axis_name="s")

    @functools.partial(
        pl.kernel, mesh=mesh,
        out_type=jax.ShapeDtypeStruct((n_half * _NHASH * _S, 2 * _DH),
                                      jnp.float32),
        scratch_types=[pltpu.VMEM((n_units * _NHASH, chunk), jnp.int32),
                       pltpu.VMEM((chunk, 2 * _DH), jnp.float32),
                       pltpu.VMEM((chunk, 2 * _DH), jnp.float32),
                       pltpu.SemaphoreType.DMA,
                       pltpu.SemaphoreType.DMA,
                       pltpu.SemaphoreType.DMA,
                       pltpu.SemaphoreType.DMA,
                       pltpu.SemaphoreType.DMA],
    )
    def k(qkv_hbm, ridx_hbm, sort_hbm, idx_all, rb0, rb1,
          isem, ls0, ls1, ss0, ss1):
        rb = (rb0, rb1)
        ls = (ls0, ls1)
        ss = (ss0, ss1)
        wid = lax.axis_index("s") * 2 + lax.axis_index("c")

        icps = []
        for u in range(n_units):
            unit = wid * n_units + u
            n = unit // n_ch
            c = unit % n_ch
            for h in range(_NHASH):
                icps.append(pltpu.async_copy(
                    ridx_hbm.at[pl.ds((n * _NHASH + h) * _S + c * chunk, chunk)],
                    idx_all.at[u * _NHASH + h], isem))
        for cp in icps:
            cp.wait()

        loads = [None] * n_units
        scats = [None] * n_units
        for u in range(n_units):
            bi = u % 2
            if u >= 2:
                for cp in scats[u - 2]:
                    cp.wait()
            unit = wid * n_units + u
            n = unit // n_ch
            c = unit % n_ch
            loads[u] = pltpu.async_copy(
                qkv_hbm.at[pl.ds((n0 + n) * _S + c * chunk, chunk)],
                rb[bi], ls[bi])
            if u >= 1:
                loads[u - 1].wait()
                scats[u - 1] = [
                    pltpu.async_copy(rb[(u - 1) % 2],
                                     sort_hbm.at[idx_all.at[(u - 1) * _NHASH + h]],
                                     ss[(u - 1) % 2])
                    for h in range(_NHASH)]
        loads[n_units - 1].wait()
        scats[n_units - 1] = [
            pltpu.async_copy(rb[(n_units - 1) % 2],
                             sort_hbm.at[idx_all.at[(n_units - 1) * _NHASH + h]],
                             ss[(n_units - 1) % 2])
            for h in range(_NHASH)]
        for u in (n_units - 2, n_units - 1):
            for cp in scats[u]:
                cp.wait()

    return k(qkv, bwd_idx)


_NHALF = _N // _B


_SB = 256
_NSB = _B * _S // _SB


def _ln_block(x, g, b):
    m = jnp.mean(x, axis=1, keepdims=True)
    v = jnp.mean((x - m) ** 2, axis=1, keepdims=True)
    return (x - m) / jnp.sqrt(v + 1e-5) * g + b


def _t_embed_ln(emb_rows, g, b):
    pe = jnp.asarray(_PE)

    def body(e_ref, p_ref, g_ref, b_ref, o_ref, oln_ref):
        x = e_ref[...] + p_ref[...]
        o_ref[...] = x
        oln_ref[...] = _ln_block(x, g_ref[...], b_ref[...])

    return pl.pallas_call(
        body,
        grid=(_NSB,),
        in_specs=[
            pl.BlockSpec((_SB, _D), lambda i: (i, 0)),
            pl.BlockSpec((_SB, _D), lambda i: (i % (_S // _SB), 0)),
            pl.BlockSpec((1, _D), lambda i: (0, 0)),
            pl.BlockSpec((1, _D), lambda i: (0, 0)),
        ],
        out_specs=[pl.BlockSpec((_SB, _D), lambda i: (i, 0)),
                   pl.BlockSpec((_SB, _D), lambda i: (i, 0))],
        out_shape=[jax.ShapeDtypeStruct((_B * _S, _D), jnp.float32),
                   jax.ShapeDtypeStruct((_B * _S, _D), jnp.float32)],
    )(emb_rows, pe, g.reshape(1, _D), b.reshape(1, _D))


def _t_qkv(xln, src_col, wqk_t, wv_t, r_flat):
    nb = _S // _SB

    def body(x_ref, s_ref, wq_ref, wv_ref, r_ref, o_ref, rot_ref):
        x = x_ref[0]
        pad = s_ref[0] == 0
        for hd in range(_H):
            wq = wq_ref[hd * _DH:(hd + 1) * _DH, :]
            wv = wv_ref[hd * _DH:(hd + 1) * _DH, :]
            qk = lax.dot_general(x, wq, (((1,), (1,)), ((), ())),
                                 preferred_element_type=jnp.float32)
            v = lax.dot_general(x, wv, (((1,), (1,)), ((), ())),
                                preferred_element_type=jnp.float32)
            rot = jnp.dot(qk, r_ref[...], preferred_element_type=jnp.float32)
            pm = []
            for h in range(_NHASH):
                rh = rot[:, h * 16:(h + 1) * 16]
                pm.extend([rh, -rh])
            rot_ref[0, hd] = jnp.concatenate(pm, axis=1)
            qkz = jnp.where(pad, 0.0, qk)
            vz = jnp.where(pad, 0.0, v)
            o_ref[0, hd] = jnp.concatenate([qkz, vz], axis=1)

    return pl.pallas_call(
        body,
        grid=(_NSB,),
        in_specs=[
            pl.BlockSpec((1, _SB, _D), lambda m: (m, 0, 0)),
            pl.BlockSpec((1, _SB, 1), lambda m: (m, 0, 0)),
            pl.BlockSpec((_D, _D), lambda m: (0, 0)),
            pl.BlockSpec((_D, _D), lambda m: (0, 0)),
            pl.BlockSpec((_DH, _DH), lambda m: (0, 0)),
        ],
        out_specs=[
            pl.BlockSpec((1, _H, _SB, 2 * _DH),
                         lambda m: (m // nb, 0, m % nb, 0)),
            pl.BlockSpec((1, _H, _SB, _NHASH * _NBKT),
                         lambda m: (m // nb, 0, m % nb, 0)),
        ],
        out_shape=[
            jax.ShapeDtypeStruct((_B, _H, _S, 2 * _DH), jnp.float32),
            jax.ShapeDtypeStruct((_B, _H, _S, _NHASH * _NBKT), jnp.float32),
        ],
    )(xln.reshape(_NSB, _SB, _D), src_col, wqk_t, wv_t, r_flat)


def _t_rank(rot):
    W = _NHASH * _NBKT

    def body(rot_ref, o_ref):
        n = pl.program_id(0)
        pm = rot_ref[0]
        lane = lax.broadcasted_iota(jnp.int32, (_S, _NBKT), 1)
        ohs = []
        for h in range(_NHASH):
            pmh = pm[:, h * _NBKT:(h + 1) * _NBKT]
            mx = jnp.max(pmh, axis=1, keepdims=True)
            amc = jnp.min(jnp.where(pmh == mx, lane, _NBKT + 1), axis=1,
                          keepdims=True)
            ohs.append((lane == amc).astype(jnp.float32))
        oh = jnp.concatenate(ohs, axis=1)
        inc = oh
        k = 1
        while k < _S:
            shifted = jnp.concatenate(
                [jnp.zeros((k, W), jnp.float32), inc[: _S - k]], axis=0)
            inc = inc + shifted
            k *= 2
        excl = inc - oh
        tot = inc[_S - 1:_S, :]
        l0 = lax.broadcasted_iota(jnp.int32, (W, W), 0)
        l1 = lax.broadcasted_iota(jnp.int32, (W, W), 1)
        ut = ((l0 // _NBKT == l1 // _NBKT) & (l0 < l1)).astype(jnp.float32)
        offs = jnp.dot(tot, ut, preferred_element_type=jnp.float32)
        prod = oh * (excl + offs)
        n_loc = n % _NHALF
        for h in range(_NHASH):
            rank = jnp.sum(prod[:, h * _NBKT:(h + 1) * _NBKT], axis=1,
                           keepdims=True)
            o_ref[0, h] = rank.astype(jnp.int32) + (n_loc * _NHASH + h) * _S

    return pl.pallas_call(
        body,
        grid=(_N,),
        in_specs=[pl.BlockSpec((1, _S, W), lambda n: (n, 0, 0))],
        out_specs=pl.BlockSpec((1, _NHASH, _S, 1), lambda n: (n, 0, 0, 0)),
        out_shape=jax.ShapeDtypeStruct((_N, _NHASH, _S, 1), jnp.int32),
    )(rot)


def _t_attention(sorted_rows):
    scale = 1.0 / math.sqrt(_DH)

    QW = 2 * _BKT
    KW = 4 * _BKT

    def body(rows_ref, o_ref):
        r_i = lax.broadcasted_iota(jnp.int32, (QW, KW), 0)
        j_i = lax.broadcasted_iota(jnp.int32, (QW, KW), 1)
        self_mask = j_i == r_i + _BKT
        rq = r_i // _BKT
        rk = j_i // _BKT
        in_window = (rk == rq) | (rk == rq + 1)
        bias = jnp.where(self_mask, -1e5,
                         jnp.where(in_window, 0.0, -1e30))
        for t in range(_NCHUNK // 2):
            p = 2 * t
            cs = p * _BKT
            if t == 0:
                win = jnp.concatenate(
                    [rows_ref[0, (_S - _BKT):_S, :], rows_ref[0, 0:3 * _BKT, :]],
                    axis=0)
            elif t == _NCHUNK // 2 - 1:
                win = jnp.concatenate(
                    [rows_ref[0, cs - _BKT:_S, :], rows_ref[0, 0:_BKT, :]],
                    axis=0)
            else:
                win = rows_ref[0, cs - _BKT:cs + 3 * _BKT, :]
            q = rows_ref[0, cs:cs + QW, 0:_DH]
            kqk = win[:, 0:_DH]
            vv = win[:, _DH:2 * _DH]
            nrm = jnp.sqrt(jnp.sum(kqk * kqk, axis=1, keepdims=True))
            kn = kqk / (nrm + 1e-9)
            dots = lax.dot_general(q, kn, (((1,), (1,)), ((), ())),
                                   preferred_element_type=jnp.float32) * scale
            padk = jnp.sum(jnp.abs(dots), axis=0, keepdims=True) == 0.0
            dots = jnp.where(padk, -1e9, dots + bias)
            mx = jnp.max(dots, axis=1, keepdims=True)
            e = jnp.exp(dots - mx)
            ssum = jnp.sum(e, axis=1, keepdims=True)
            lse = mx + jnp.log(ssum)
            o = jnp.dot(e, vv, preferred_element_type=jnp.float32) / ssum
            o_ref[0, cs:cs + QW, 0:_DH] = o
            o_ref[0, cs:cs + QW, _DH:_DH + 1] = lse

    ng = sorted_rows.shape[0]
    return pl.pallas_call(
        body,
        grid=(ng,),
        in_specs=[pl.BlockSpec((1, _S, 2 * _DH), lambda g: (g, 0, 0))],
        out_specs=pl.BlockSpec((1, _S, 2 * _DH), lambda g: (g, 0, 0)),
        out_shape=jax.ShapeDtypeStruct((ng, _S, 2 * _DH), jnp.float32),
    )(sorted_rows)


def _t_combine(o_unsorted, x_res_b, wo, g, b):
    nb = _S // _SB

    def body(o_ref, x_ref, wo_ref, g_ref, b_ref, out_ref, oln_ref):
        parts = []
        for hd in range(_H):
            ls = [o_ref[hd * _NHASH + j, :, _DH:_DH + 1]
                  for j in range(_NHASH)]
            os_ = [o_ref[hd * _NHASH + j, :, 0:_DH] for j in range(_NHASH)]
            mx = jnp.maximum(jnp.maximum(ls[0], ls[1]),
                             jnp.maximum(ls[2], ls[3]))
            ws = [jnp.exp(l - mx) for l in ls]
            tot = ws[0] + ws[1] + ws[2] + ws[3]
            ctx = (ws[0] * os_[0] + ws[1] * os_[1]
                   + ws[2] * os_[2] + ws[3] * os_[3]) / tot
            parts.append(ctx)
        ctx = jnp.concatenate(parts, axis=1)
        x1 = x_ref[0] + jnp.dot(ctx, wo_ref[...],
                                preferred_element_type=jnp.float32)
        out_ref[0] = x1
        oln_ref[0] = _ln_block(x1, g_ref[...], b_ref[...])

    return pl.pallas_call(
        body,
        grid=(nb,),
        in_specs=[
            pl.BlockSpec((_H * _NHASH, _SB, 2 * _DH), lambda s: (0, s, 0)),
            pl.BlockSpec((1, _SB, _D), lambda s: (s, 0, 0)),
            pl.BlockSpec((_D, _D), lambda s: (0, 0)),
            pl.BlockSpec((1, _D), lambda s: (0, 0)),
            pl.BlockSpec((1, _D), lambda s: (0, 0)),
        ],
        out_specs=[
            pl.BlockSpec((1, _SB, _D), lambda s: (s, 0, 0)),
            pl.BlockSpec((1, _SB, _D), lambda s: (s, 0, 0)),
        ],
        out_shape=[jax.ShapeDtypeStruct((nb, _SB, _D), jnp.float32),
                   jax.ShapeDtypeStruct((nb, _SB, _D), jnp.float32)],
    )(o_unsorted.reshape(_H * _NHASH, _S, 2 * _DH),
      x_res_b.reshape(nb, _SB, _D), wo, g.reshape(1, _D), b.reshape(1, _D))


def _t_ffn1(xln, w1, b1):
    def body(x_ref, w_ref, b1_ref, o_ref):
        o_ref[0] = jax.nn.gelu(
            jnp.dot(x_ref[0], w_ref[...], preferred_element_type=jnp.float32)
            + b1_ref[...])

    return pl.pallas_call(
        body,
        grid=(_NSB,),
        in_specs=[
            pl.BlockSpec((1, _SB, _D), lambda m: (m, 0, 0)),
            pl.BlockSpec((_D, 4 * _D), lambda m: (0, 0)),
            pl.BlockSpec((1, 4 * _D), lambda m: (0, 0)),
        ],
        out_specs=pl.BlockSpec((1, _SB, 4 * _D), lambda m: (m, 0, 0)),
        out_shape=jax.ShapeDtypeStruct((_NSB, _SB, 4 * _D), jnp.float32),
    )(xln, w1, b1.reshape(1, 4 * _D))


def _t_ffn2(h1, x_res, w2, b2, lng=None, lnb=None):
    with_ln = lng is not None

    def body(h_ref, x_ref, w_ref, b_ref, *rest):
        x = (x_ref[0]
             + jnp.dot(h_ref[0], w_ref[...], preferred_element_type=jnp.float32)
             + b_ref[...])
        if with_ln:
            g_ref, lb_ref, o_ref, oln_ref = rest
            o_ref[0] = x
            oln_ref[0] = _ln_block(x, g_ref[...], lb_ref[...])
        else:
            (o_ref,) = rest
            o_ref[0] = x

    in_specs = [
        pl.BlockSpec((1, _SB, 4 * _D), lambda m: (m, 0, 0)),
        pl.BlockSpec((1, _SB, _D), lambda m: (m, 0, 0)),
        pl.BlockSpec((4 * _D, _D), lambda m: (0, 0)),
        pl.BlockSpec((1, _D), lambda m: (0, 0)),
    ]
    args = [h1, x_res.reshape(_NSB, _SB, _D), w2, b2.reshape(1, _D)]
    out_spec = pl.BlockSpec((1, _SB, _D), lambda m: (m, 0, 0))
    if with_ln:
        in_specs += [pl.BlockSpec((1, _D), lambda m: (0, 0))] * 2
        args += [lng.reshape(1, _D), lnb.reshape(1, _D)]
        out_specs = [out_spec, out_spec]
        out_shape = [jax.ShapeDtypeStruct((_NSB, _SB, _D), jnp.float32)] * 2
    else:
        out_specs = out_spec
        out_shape = jax.ShapeDtypeStruct((_NSB, _SB, _D), jnp.float32)

    return pl.pallas_call(
        body,
        grid=(_NSB,),
        in_specs=in_specs,
        out_specs=out_specs,
        out_shape=out_shape,
    )(*args)


def _t_classifier(x, src_col, wp, bp, wc_pad, bc_pad):
    def body(x_ref, s_ref, wp_ref, bp_ref, wc_ref, bc_ref, o_ref):
        keep = (s_ref[0] != 0).astype(jnp.float32)
        hidden = x_ref[0] * keep
        summed = jnp.sum(hidden, axis=0, keepdims=True)
        cnt = jnp.sum(keep, axis=0, keepdims=True)
        pooled = summed / cnt
        pr = jnp.maximum(
            jnp.dot(pooled, wp_ref[...], preferred_element_type=jnp.float32)
            + bp_ref[...], 0.0)
        o_ref[0] = (jnp.dot(pr, wc_ref[...], preferred_element_type=jnp.float32)
                    + bc_ref[...])

    return pl.pallas_call(
        body,
        grid=(_B,),
        in_specs=[
            pl.BlockSpec((1, _S, _D), lambda b: (b, 0, 0)),
            pl.BlockSpec((1, _S, 1), lambda b: (b, 0, 0)),
            pl.BlockSpec((_D, _D), lambda b: (0, 0)),
            pl.BlockSpec((1, _D), lambda b: (0, 0)),
            pl.BlockSpec((_D, 64), lambda b: (0, 0)),
            pl.BlockSpec((1, 64), lambda b: (0, 0)),
        ],
        out_specs=pl.BlockSpec((1, 1, 64), lambda b: (b, 0, 0)),
        out_shape=jax.ShapeDtypeStruct((_B, 1, 64), jnp.float32),
    )(x.reshape(_B, _S, _D), src_col, wp, bp.reshape(1, _D), wc_pad, bc_pad)


def _layer(x, xln, p, src_col, next_ln):
    wqk_t = p['Wqk'].T
    wv_t = p['Wv'].T
    r_flat = p['rotations'].reshape(_DH, _DH)
    qkv, rot = _t_qkv(xln, src_col, wqk_t, wv_t, r_flat)
    bwd_idx = _t_rank(rot.reshape(_N, _S, _NHASH * _NBKT))
    bwd_halves = bwd_idx.reshape(_B, _NHALF * _NHASH * _S)
    qkv_flat = qkv.reshape(_N * _S, 2 * _DH)
    gh = _NHALF * _NHASH
    x1s, xln2s = [], []
    for half in range(_B):
        bwd_h = bwd_halves[half]
        sorted_h = _sc_scatter_sorted(qkv_flat, bwd_h, half * _NHALF, _NHALF)
        so_h = _t_attention(sorted_h.reshape(gh, _S, 2 * _DH))
        ou_h = _sc_gather_rows(so_h.reshape(gh * _S, 2 * _DH), bwd_h)
        x1_h, xln2_h = _t_combine(ou_h, x.reshape(_B, _S, _D)[half],
                                  p['Wo'], p['ln2_g'], p['ln2_b'])
        x1s.append(x1_h.reshape(_S, _D))
        xln2s.append(xln2_h.reshape(_S, _D))
    x1 = jnp.concatenate(x1s, axis=0)
    xln2 = jnp.concatenate(xln2s, axis=0).reshape(_NSB, _SB, _D)
    h1 = _t_ffn1(xln2, p['W1'], p['b1f'])
    if next_ln is None:
        x2 = _t_ffn2(h1, x1, p['W2'], p['b2f'])
        return x2.reshape(_B * _S, _D), None
    x2, xlnn = _t_ffn2(h1, x1, p['W2'], p['b2f'], next_ln[0], next_ln[1])
    return x2.reshape(_B * _S, _D), xlnn


def kernel(src, source_lengths, params):
    del source_lengths
    src = src.astype(jnp.int32)
    emb_rows = _sc_gather_rows(params['emb'], src.reshape(_B * _S))
    layers = params['layers']
    x, xln = _t_embed_ln(emb_rows, layers[0]['ln1_g'], layers[0]['ln1_b'])
    src_col = src.reshape(_NSB, _SB, 1)
    for li, p in enumerate(layers):
        nxt = None
        if li + 1 < len(layers):
            nxt = (layers[li + 1]['ln1_g'], layers[li + 1]['ln1_b'])
        x, xln = _layer(x, xln, p, src_col, nxt)
    wc_pad = jnp.pad(params['Wc'], ((0, 0), (0, 64 - _NCLS)))
    bc_pad = jnp.pad(params['bc'], (0, 64 - _NCLS)).reshape(1, 64)
    logits = _t_classifier(x, src.reshape(_B, _S, 1), params['Wp'],
                           params['bp'], wc_pad, bc_pad)
    return logits.reshape(_B, 64)[:, :_NCLS]
